# baseline (speedup 1.0000x reference)
import functools

import jax
import jax.numpy as jnp
from jax import lax
from jax.experimental import pallas as pl
from jax.experimental.pallas import tpu as pltpu
from jax.experimental.pallas import tpu_sc as plsc

VOCAB = 1000000
SENT_LEN = 200
DIM = 64
BATCH = 4096

NW = 32
ROWS = BATCH * SENT_LEN
CHUNK = 128
ROWS_W = ROWS // NW
NCHUNK = ROWS_W // CHUNK
NBUF = 5
LOOKAHEAD = 3
NLANE = 16
COLS = DIM // NLANE

_mesh = plsc.VectorSubcoreMesh(core_axis_name="c", subcore_axis_name="s")


@functools.partial(
    pl.kernel,
    out_type=jax.ShapeDtypeStruct((ROWS, DIM), jnp.float32),
    mesh=_mesh,
    scratch_types=[
        pltpu.VMEM((NCHUNK, CHUNK), jnp.int32),
        pltpu.VMEM((2 * SENT_LEN, DIM), jnp.float32),
    ]
    + [pltpu.VMEM((CHUNK, DIM), jnp.float32) for _ in range(NBUF)]
    + [pltpu.SemaphoreType.DMA for _ in range(2 * NBUF)],
    compiler_params=pltpu.CompilerParams(use_tc_tiling_on_sc=False),
)
def _emb_lookup(idx_hbm, tok_hbm, pos_hbm, out_hbm, idx_v, pos_v, *bufs_sems):
    rows_v = list(bufs_sems[:NBUF])
    gsem = list(bufs_sems[NBUF:2 * NBUF])
    ssem = list(bufs_sems[2 * NBUF:])

    wid = lax.axis_index("s") * 2 + lax.axis_index("c")
    row0 = wid * ROWS_W
    crow0 = wid * NCHUNK

    pltpu.sync_copy(idx_hbm.at[pl.ds(crow0, NCHUNK)], idx_v)
    pltpu.sync_copy(pos_hbm, pos_v)

    def issue_gather(c, b):
        pltpu.async_copy(tok_hbm.at[idx_v.at[c]], rows_v[b], gsem[b])

    def wait_gather(b):
        pltpu.make_async_copy(tok_hbm.at[pl.ds(0, CHUNK)], rows_v[b], gsem[b]).wait()

    def issue_scatter(c, b):
        pltpu.async_copy(rows_v[b], out_hbm.at[pl.ds(row0 + c * CHUNK, CHUNK)],
                         ssem[b])

    def wait_scatter(b):
        pltpu.make_async_copy(rows_v[b], out_hbm.at[pl.ds(0, CHUNK)],
                              ssem[b]).wait()

    def add_positions(c, b):
        if isinstance(c, int):
            p0 = (c * CHUNK) % SENT_LEN
        else:
            p0 = lax.rem(c * CHUNK, SENT_LEN)
        buf = rows_v[b]

        def body(i, carry):
            for j in range(COLS):
                sl = pl.ds(j * NLANE, NLANE)
                buf[i, sl] = buf[i, sl] + pos_v[p0 + i, sl]
            return carry

        lax.fori_loop(0, CHUNK, body, 0, unroll=2)

    def stage(c, b, *, wait_sc, issue_g):
        bg = (b + LOOKAHEAD) % NBUF
        wait_gather(b)
        add_positions(c, b)
        issue_scatter(c, b)
        if wait_sc:
            wait_scatter(bg)
        if issue_g:
            issue_gather(c + LOOKAHEAD, bg)

    for c in range(LOOKAHEAD):
        issue_gather(c, c)

    for b in range(NBUF):
        stage(b, b, wait_sc=(b >= 2), issue_g=True)

    def outer(c5, carry):
        for b in range(NBUF):
            stage(c5 * NBUF + b, b, wait_sc=True, issue_g=True)
        return carry

    lax.fori_loop(1, NCHUNK // NBUF - 1, outer, 0)

    for b in range(NBUF):
        c = NCHUNK - NBUF + b
        stage(c, b, wait_sc=True, issue_g=(b < NBUF - LOOKAHEAD))

    for c in range(NCHUNK - (NBUF - LOOKAHEAD), NCHUNK):
        wait_scatter(c % NBUF)


def kernel(inputs, token_table, position_table):
    idx2d = inputs.reshape(ROWS // CHUNK, CHUNK).astype(jnp.int32)
    pos2 = jnp.concatenate([position_table, position_table], axis=0)
    out = _emb_lookup(idx2d, token_table, pos2)
    return out.reshape(BATCH, SENT_LEN, DIM)


# speedup vs baseline: 1.2063x; 1.2063x over previous
import functools

import jax
import jax.numpy as jnp
from jax import lax
from jax.experimental import pallas as pl
from jax.experimental.pallas import tpu as pltpu
from jax.experimental.pallas import tpu_sc as plsc

VOCAB = 1000000
SENT_LEN = 200
DIM = 64
BATCH = 4096

NW = 32
CHUNK = 128
NCHUNK = SENT_LEN
NBUF = 4
LOOKAHEAD = 3
NLANE = 16
DTILE = DIM // 8

_mesh = plsc.VectorSubcoreMesh(core_axis_name="c", subcore_axis_name="s")


@functools.partial(
    pl.kernel,
    out_type=jax.ShapeDtypeStruct((SENT_LEN, DTILE, NW, 8, 128), jnp.float32),
    mesh=_mesh,
    scratch_types=[
        pltpu.VMEM((NCHUNK, CHUNK), jnp.int32),
        pltpu.VMEM((SENT_LEN, DIM), jnp.float32),
    ]
    + [pltpu.VMEM((CHUNK, DIM), jnp.float32) for _ in range(NBUF)]
    + [pltpu.VMEM((DTILE, 8, 128), jnp.float32) for _ in range(NBUF)]
    + [pltpu.SemaphoreType.DMA for _ in range(2 * NBUF)],
    compiler_params=pltpu.CompilerParams(
        use_tc_tiling_on_sc=False, needs_layout_passes=False),
)
def _emb_lookup(idx_hbm, tok_hbm, pos_hbm, out_hbm, idx_v, pos_v, *bufs_sems):
    gbuf = list(bufs_sems[:NBUF])
    tbuf = list(bufs_sems[NBUF:2 * NBUF])
    gsem = list(bufs_sems[2 * NBUF:3 * NBUF])
    ssem = list(bufs_sems[3 * NBUF:])

    wid = lax.axis_index("s") * 2 + lax.axis_index("c")

    pltpu.sync_copy(idx_hbm.at[:, pl.ds(wid * CHUNK, CHUNK)], idx_v)
    pltpu.sync_copy(pos_hbm, pos_v)

    iota = lax.iota(jnp.int32, NLANE)

    def issue_gather(c, b):
        pltpu.async_copy(tok_hbm.at[idx_v.at[c]], gbuf[b], gsem[b])

    def wait_gather(b):
        pltpu.make_async_copy(tok_hbm.at[pl.ds(0, CHUNK)], gbuf[b], gsem[b]).wait()

    def issue_scatter(c, b):
        pltpu.async_copy(tbuf[b], out_hbm.at[c, :, wid], ssem[b])

    def wait_scatter(b):
        pltpu.make_async_copy(tbuf[b], out_hbm.at[0, :, 0], ssem[b]).wait()

    def transpose_add(c, b):
        src = gbuf[b]
        dst = tbuf[b]
        srow = jnp.full((NLANE,), c, jnp.int32)

        @plsc.parallel_loop(0, DIM, unroll=2)
        def _(d):
            dr = lax.shift_right_logical(d, 3)
            di = lax.bitwise_and(d, 7)
            colv = jnp.full((NLANE,), d, jnp.int32)
            posv = plsc.load_gather(pos_v, [srow, colv])
            for k in range(CHUNK // NLANE):
                rowv = iota + (k * NLANE)
                vals = plsc.load_gather(src, [rowv, colv])
                dst[dr, di, pl.ds(k * NLANE, NLANE)] = vals + posv

    def stage(c, b, *, wait_sc, issue_g):
        wait_gather(b)
        if wait_sc:
            wait_scatter(b)
        transpose_add(c, b)
        issue_scatter(c, b)
        if issue_g:
            issue_gather(c + LOOKAHEAD, (b + LOOKAHEAD) % NBUF)

    for c in range(LOOKAHEAD):
        issue_gather(c, c)

    for b in range(NBUF):
        stage(b, b, wait_sc=False, issue_g=True)

    def outer(c4, carry):
        for b in range(NBUF):
            stage(c4 * NBUF + b, b, wait_sc=True, issue_g=True)
        return carry

    lax.fori_loop(1, NCHUNK // NBUF - 1, outer, 0)

    for b in range(NBUF):
        c = NCHUNK - NBUF + b
        stage(c, b, wait_sc=True, issue_g=(c + LOOKAHEAD < NCHUNK))

    for b in range(NBUF):
        wait_scatter(b)


def kernel(inputs, token_table, position_table):
    idx_t = inputs.T.astype(jnp.int32)
    out5 = _emb_lookup(idx_t, token_table, position_table)
    return out5.transpose(2, 4, 0, 1, 3).reshape(BATCH, SENT_LEN, DIM)


# speedup vs baseline: 2.0677x; 1.7140x over previous
import functools

import jax
import jax.numpy as jnp
from jax import lax
from jax.experimental import pallas as pl
from jax.experimental.pallas import tpu as pltpu
from jax.experimental.pallas import tpu_sc as plsc

VOCAB = 1000000
SENT_LEN = 200
DIM = 64
BATCH = 4096

NW = 32
CHUNK = 128
NCHUNK = SENT_LEN
NBUF = 4
LOOKAHEAD = 3
NLANE = 16
DTILE = DIM // 8

_mesh = plsc.VectorSubcoreMesh(core_axis_name="c", subcore_axis_name="s")


@functools.partial(
    pl.kernel,
    out_type=jax.ShapeDtypeStruct((SENT_LEN, DTILE, NW, 8, 128), jnp.float32),
    mesh=_mesh,
    scratch_types=[
        pltpu.VMEM((NCHUNK, CHUNK), jnp.int32),
        pltpu.VMEM((SENT_LEN, DIM), jnp.float32),
    ]
    + [pltpu.VMEM((CHUNK, DIM), jnp.float32) for _ in range(NBUF)]
    + [pltpu.VMEM((DTILE, 8, 129), jnp.float32) for _ in range(NBUF)]
    + [pltpu.SemaphoreType.DMA for _ in range(2 * NBUF)],
    compiler_params=pltpu.CompilerParams(
        use_tc_tiling_on_sc=False, needs_layout_passes=False),
)
def _emb_lookup(idx_hbm, tok_hbm, pos_hbm, out_hbm, idx_v, pos_v, *bufs_sems):
    gbuf = list(bufs_sems[:NBUF])
    tbuf = list(bufs_sems[NBUF:2 * NBUF])
    gsem = list(bufs_sems[2 * NBUF:3 * NBUF])
    ssem = list(bufs_sems[3 * NBUF:])

    wid = lax.axis_index("s") * 2 + lax.axis_index("c")

    pltpu.sync_copy(idx_hbm.at[:, pl.ds(wid * CHUNK, CHUNK)], idx_v)
    pltpu.sync_copy(pos_hbm, pos_v)

    iota = lax.iota(jnp.int32, NLANE)

    def issue_gather(c, b):
        pltpu.async_copy(tok_hbm.at[idx_v.at[c]], gbuf[b], gsem[b])

    def wait_gather(b):
        pltpu.make_async_copy(tok_hbm.at[pl.ds(0, CHUNK)], gbuf[b], gsem[b]).wait()

    def issue_scatter(c, b):
        pltpu.async_copy(tbuf[b].at[:, :, pl.ds(0, 128)],
                         out_hbm.at[c, :, wid], ssem[b])

    def wait_scatter(b):
        pltpu.make_async_copy(tbuf[b].at[:, :, pl.ds(0, 128)],
                              out_hbm.at[0, :, 0], ssem[b]).wait()

    dr_vecs = [lax.shift_right_logical(iota + j * NLANE, 3)
               for j in range(DIM // NLANE)]
    di_vecs = [lax.bitwise_and(iota + j * NLANE, 7)
               for j in range(DIM // NLANE)]

    def transpose_add(c, b):
        src = gbuf[b]
        dst = tbuf[b]
        posv = [pos_v[c, pl.ds(j * NLANE, NLANE)] for j in range(DIM // NLANE)]

        @plsc.parallel_loop(0, CHUNK, unroll=2)
        def _(bb):
            bsplat = jnp.full((NLANE,), bb, jnp.int32)
            for j in range(DIM // NLANE):
                vals = src[bb, pl.ds(j * NLANE, NLANE)] + posv[j]
                plsc.store_scatter(dst, [dr_vecs[j], di_vecs[j], bsplat], vals)

    def stage(c, b, *, wait_sc, issue_g):
        wait_gather(b)
        if wait_sc:
            wait_scatter(b)
        transpose_add(c, b)
        issue_scatter(c, b)
        if issue_g:
            issue_gather(c + LOOKAHEAD, (b + LOOKAHEAD) % NBUF)

    for c in range(LOOKAHEAD):
        issue_gather(c, c)

    for b in range(NBUF):
        stage(b, b, wait_sc=False, issue_g=True)

    def outer(c4, carry):
        for b in range(NBUF):
            stage(c4 * NBUF + b, b, wait_sc=True, issue_g=True)
        return carry

    lax.fori_loop(1, NCHUNK // NBUF - 1, outer, 0)

    for b in range(NBUF):
        c = NCHUNK - NBUF + b
        stage(c, b, wait_sc=True, issue_g=(c + LOOKAHEAD < NCHUNK))

    for b in range(NBUF):
        wait_scatter(b)


def kernel(inputs, token_table, position_table):
    idx_t = inputs.T.astype(jnp.int32)
    out5 = _emb_lookup(idx_t, token_table, position_table)
    return out5.transpose(2, 4, 0, 1, 3).reshape(BATCH, SENT_LEN, DIM)
